# Initial kernel scaffold; baseline (speedup 1.0000x reference)
import jax
import jax.numpy as jnp
from jax.experimental import pallas as pl


def kernel(rate, U, I, u_index, s_index):
    raise NotImplementedError("write your pallas kernel here")


# speedup vs baseline: 3.6180x; 3.6180x over previous
import functools

import jax
import jax.numpy as jnp
from jax import lax
from jax.experimental import pallas as pl
from jax.experimental.pallas import tpu as pltpu
from jax.experimental.pallas import tpu_sc as plsc

_NC = 2
_NS = 16
_NW = _NC * _NS

_B = 16384
_D = 64
_BPW = _B // _NW


def _mse_body(rate_hbm, uidx_hbm, sidx_hbm, u_hbm, i_hbm, out_hbm,
              uidx_v, sidx_v, rate_v, urows_v, irows_v, part_v,
              sem_u, sem_i):
    wid = lax.axis_index("s") * _NC + lax.axis_index("c")
    base = wid * _BPW

    pltpu.sync_copy(uidx_hbm.at[pl.ds(base, _BPW)], uidx_v)
    pltpu.sync_copy(sidx_hbm.at[pl.ds(base, _BPW)], sidx_v)
    pltpu.sync_copy(rate_hbm.at[pl.ds(base, _BPW)], rate_v)

    cu = pltpu.async_copy(u_hbm.at[uidx_v], urows_v, sem_u)
    ci = pltpu.async_copy(i_hbm.at[sidx_v], irows_v, sem_i)
    cu.wait()
    ci.wait()

    lane = lax.iota(jnp.int32, 16)
    perms = [(lane ^ sh).astype(jnp.int32) for sh in (1, 2, 4, 8)]
    _dnums = lax.GatherDimensionNumbers(
        offset_dims=(), collapsed_slice_dims=(0,), start_index_map=(0,))

    def _shuffle(vec, perm):
        return lax.gather(vec, perm[:, None], _dnums, slice_sizes=(1,),
                          mode=lax.GatherScatterMode.PROMISE_IN_BOUNDS)

    def chunk(k, tot16):
        r0 = k * 16
        rate16 = rate_v[pl.ds(r0, 16)]
        pred16 = jnp.zeros((16,), jnp.float32)
        for j in range(16):
            r = r0 + j
            p = urows_v[r, pl.ds(0, 16)] * irows_v[r, pl.ds(0, 16)]
            for q in range(1, _D // 16):
                p = p + (urows_v[r, pl.ds(16 * q, 16)]
                         * irows_v[r, pl.ds(16 * q, 16)])
            for perm in perms:
                p = p + _shuffle(p, perm)
            pred16 = jnp.where(lane == j, p, pred16)
        dlt = pred16 - rate16
        return tot16 + dlt * dlt

    tot16 = lax.fori_loop(0, _BPW // 16, chunk, jnp.zeros((16,), jnp.float32))

    part_v[...] = tot16
    pltpu.sync_copy(part_v, out_hbm.at[wid])


@functools.partial(
    pl.kernel,
    out_type=jax.ShapeDtypeStruct((_NW, 16), jnp.float32),
    mesh=plsc.VectorSubcoreMesh(core_axis_name="c", subcore_axis_name="s"),
    compiler_params=pltpu.CompilerParams(use_tc_tiling_on_sc=False),
    scratch_types=[
        pltpu.VMEM((_BPW,), jnp.int32),
        pltpu.VMEM((_BPW,), jnp.int32),
        pltpu.VMEM((_BPW,), jnp.float32),
        pltpu.VMEM((_BPW, _D), jnp.float32),
        pltpu.VMEM((_BPW, _D), jnp.float32),
        pltpu.VMEM((16,), jnp.float32),
        pltpu.SemaphoreType.DMA,
        pltpu.SemaphoreType.DMA,
    ],
)
def _mse_partials(rate_hbm, uidx_hbm, sidx_hbm, u_hbm, i_hbm, out_hbm,
                  uidx_v, sidx_v, rate_v, urows_v, irows_v, part_v,
                  sem_u, sem_i):
    _mse_body(rate_hbm, uidx_hbm, sidx_hbm, u_hbm, i_hbm, out_hbm,
              uidx_v, sidx_v, rate_v, urows_v, irows_v, part_v,
              sem_u, sem_i)


def kernel(rate, U, I, u_index, s_index):
    parts = _mse_partials(rate, u_index.astype(jnp.int32),
                          s_index.astype(jnp.int32), U, I)
    return jnp.sum(parts) * jnp.float32(1.0 / _B)


# speedup vs baseline: 3.7442x; 1.0349x over previous
import functools

import jax
import jax.numpy as jnp
from jax import lax
from jax.experimental import pallas as pl
from jax.experimental.pallas import tpu as pltpu
from jax.experimental.pallas import tpu_sc as plsc

_NC = 2
_NS = 16
_NW = _NC * _NS

_B = 16384
_D = 64
_BPW = _B // _NW


def _mse_body(rate_hbm, uidx_hbm, sidx_hbm, u_hbm, i_hbm, out_hbm,
              uidx_v, sidx_v, rate_v, urows_v, irows_v, part_v,
              sem_u, sem_i):
    wid = lax.axis_index("s") * _NC + lax.axis_index("c")
    base = wid * _BPW

    pltpu.sync_copy(uidx_hbm.at[pl.ds(base, _BPW)], uidx_v)
    pltpu.sync_copy(sidx_hbm.at[pl.ds(base, _BPW)], sidx_v)
    pltpu.sync_copy(rate_hbm.at[pl.ds(base, _BPW)], rate_v)

    cu = pltpu.async_copy(u_hbm.at[uidx_v], urows_v, sem_u)
    ci = pltpu.async_copy(i_hbm.at[sidx_v], irows_v, sem_i)
    cu.wait()
    ci.wait()

    lane = lax.iota(jnp.int32, 16)
    perms = [(lane ^ sh).astype(jnp.int32) for sh in (1, 2, 4, 8)]
    _dnums = lax.GatherDimensionNumbers(
        offset_dims=(), collapsed_slice_dims=(0,), start_index_map=(0,))

    def _shuffle(vec, perm):
        return lax.gather(vec, perm[:, None], _dnums, slice_sizes=(1,),
                          mode=lax.GatherScatterMode.PROMISE_IN_BOUNDS)

    def chunk(k, tot16):
        r0 = k * 16
        rate16 = rate_v[pl.ds(r0, 16)]

        def quad(k4, pred16):
            for j2 in range(4):
                j = k4 * 4 + j2
                r = r0 + j
                p = urows_v[r, pl.ds(0, 16)] * irows_v[r, pl.ds(0, 16)]
                for q in range(1, _D // 16):
                    p = p + (urows_v[r, pl.ds(16 * q, 16)]
                             * irows_v[r, pl.ds(16 * q, 16)])
                for perm in perms:
                    p = p + _shuffle(p, perm)
                pred16 = jnp.where(lane == j, p, pred16)
            return pred16

        pred16 = lax.fori_loop(0, 4, quad, jnp.zeros((16,), jnp.float32))
        dlt = pred16 - rate16
        return tot16 + dlt * dlt

    tot16 = lax.fori_loop(0, _BPW // 16, chunk, jnp.zeros((16,), jnp.float32))

    part_v[...] = tot16
    pltpu.sync_copy(part_v, out_hbm.at[wid])


@functools.partial(
    pl.kernel,
    out_type=jax.ShapeDtypeStruct((_NW, 16), jnp.float32),
    mesh=plsc.VectorSubcoreMesh(core_axis_name="c", subcore_axis_name="s"),
    compiler_params=pltpu.CompilerParams(use_tc_tiling_on_sc=False),
    scratch_types=[
        pltpu.VMEM((_BPW,), jnp.int32),
        pltpu.VMEM((_BPW,), jnp.int32),
        pltpu.VMEM((_BPW,), jnp.float32),
        pltpu.VMEM((_BPW, _D), jnp.float32),
        pltpu.VMEM((_BPW, _D), jnp.float32),
        pltpu.VMEM((16,), jnp.float32),
        pltpu.SemaphoreType.DMA,
        pltpu.SemaphoreType.DMA,
    ],
)
def _mse_partials(rate_hbm, uidx_hbm, sidx_hbm, u_hbm, i_hbm, out_hbm,
                  uidx_v, sidx_v, rate_v, urows_v, irows_v, part_v,
                  sem_u, sem_i):
    _mse_body(rate_hbm, uidx_hbm, sidx_hbm, u_hbm, i_hbm, out_hbm,
              uidx_v, sidx_v, rate_v, urows_v, irows_v, part_v,
              sem_u, sem_i)


def kernel(rate, U, I, u_index, s_index):
    parts = _mse_partials(rate, u_index, s_index, U, I)
    return jnp.sum(parts) * jnp.float32(1.0 / _B)
